# Initial kernel scaffold; baseline (speedup 1.0000x reference)
import jax
import jax.numpy as jnp
from jax.experimental import pallas as pl


def kernel(feat, btype, Ws, bs, Wg, bg, Wl, bl):
    raise NotImplementedError("write your pallas kernel here")


# speedup vs baseline: 1.5452x; 1.5452x over previous
import functools

import jax
import jax.numpy as jnp
from jax.experimental import pallas as pl
from jax.experimental.pallas import tpu as pltpu

_BLK = 2000


def _body(feat_ref, bt_ref, w_ref, out_ref):
    f = feat_ref[...]
    t = bt_ref[...]
    f0 = f[:, 0:1]
    f1 = f[:, 1:2]
    m1 = t == 1
    m2 = t == 2
    m3 = t == 3
    B = f.shape[0]
    col = jax.lax.broadcasted_iota(jnp.int32, (B, 16), 1)
    one = jnp.ones((B, 1), jnp.float32)

    def put(c, val, mask):
        return jnp.where((col == c) & mask, val, 0.0)

    fc = (put(0, f0, m1) + put(1, f1, m1)
          + put(2, f0, m2) + put(3, f1, m2)
          + put(4, f0, m3) + put(5, f1, m3)
          + put(6, one, m1) + put(7, one, m2) + put(8, one, m3))
    pre = jnp.dot(fc, w_ref[...], preferred_element_type=jnp.float32)
    out_ref[...] = jnp.tanh(pre)


@jax.jit
def kernel(feat, btype, Ws, bs, Wg, bg, Wl, bl):
    n, _ = feat.shape
    d = Ws.shape[1]
    wcat = jnp.zeros((16, d), jnp.float32)
    wcat = wcat.at[0:2].set(Ws).at[2:4].set(Wg).at[4:6].set(Wl)
    wcat = wcat.at[6].set(bs).at[7].set(bg).at[8].set(bl)
    bt2 = btype.reshape(n, 1)

    grid = (n // _BLK,)
    return pl.pallas_call(
        _body,
        grid=grid,
        in_specs=[
            pl.BlockSpec((_BLK, 2), lambda i: (i, 0)),
            pl.BlockSpec((_BLK, 1), lambda i: (i, 0)),
            pl.BlockSpec((16, d), lambda i: (0, 0)),
        ],
        out_specs=pl.BlockSpec((_BLK, d), lambda i: (i, 0)),
        out_shape=jax.ShapeDtypeStruct((n, d), jnp.float32),
    )(feat, bt2, wcat)


# speedup vs baseline: 3.3425x; 2.1631x over previous
import functools

import jax
import jax.numpy as jnp
from jax import lax
from jax.experimental import pallas as pl
from jax.experimental.pallas import tpu as pltpu
from jax.experimental.pallas import tpu_sc as plsc

_NW = 32
_SLAB = 3200
_NPAD = _NW * _SLAB
_GROUPS = _SLAB // 16
_BLK = 2048


def _route_body(f0_hbm, f1_hbm, bt_hbm, out_hbm, f0_v, f1_v, bt_v, buf_v):
    wid = lax.axis_index("s") * 2 + lax.axis_index("c")
    base = wid * _SLAB
    pltpu.sync_copy(f0_hbm.at[pl.ds(base, _SLAB)], f0_v)
    pltpu.sync_copy(f1_hbm.at[pl.ds(base, _SLAB)], f1_v)
    pltpu.sync_copy(bt_hbm.at[pl.ds(base, _SLAB)], bt_v)

    zero_f = jnp.zeros((16,), jnp.float32)
    one_f = jnp.ones((16,), jnp.float32)

    def group(k, carry):
        r = k * 16
        tv = bt_v[pl.ds(r, 16)]
        f0 = f0_v[pl.ds(r, 16)]
        f1 = f1_v[pl.ds(r, 16)]
        m1 = tv == 1
        m2 = tv == 2
        m3 = tv == 3
        buf_v[0, pl.ds(r, 16)] = jnp.where(m1, f0, zero_f)
        buf_v[1, pl.ds(r, 16)] = jnp.where(m1, f1, zero_f)
        buf_v[2, pl.ds(r, 16)] = jnp.where(m2, f0, zero_f)
        buf_v[3, pl.ds(r, 16)] = jnp.where(m2, f1, zero_f)
        buf_v[4, pl.ds(r, 16)] = jnp.where(m3, f0, zero_f)
        buf_v[5, pl.ds(r, 16)] = jnp.where(m3, f1, zero_f)
        buf_v[6, pl.ds(r, 16)] = jnp.where(m1, one_f, zero_f)
        buf_v[7, pl.ds(r, 16)] = jnp.where(m2, one_f, zero_f)
        buf_v[8, pl.ds(r, 16)] = jnp.where(m3, one_f, zero_f)
        for c in range(9, 16):
            buf_v[c, pl.ds(r, 16)] = zero_f
        return carry

    lax.fori_loop(0, _GROUPS, group, 0)
    pltpu.sync_copy(buf_v, out_hbm.at[:, pl.ds(base, _SLAB)])


def _route(f0a, f1a, bt_pad):
    mesh = plsc.VectorSubcoreMesh(core_axis_name="c", subcore_axis_name="s")
    fn = functools.partial(
        pl.kernel,
        mesh=mesh,
        out_type=jax.ShapeDtypeStruct((16, _NPAD), jnp.float32),
        scratch_types=[
            pltpu.VMEM((_SLAB,), jnp.float32),
            pltpu.VMEM((_SLAB,), jnp.float32),
            pltpu.VMEM((_SLAB,), jnp.int32),
            pltpu.VMEM((16, _SLAB), jnp.float32),
        ],
    )(_route_body)
    return fn(f0a, f1a, bt_pad)


def _dense_body(fc_ref, w_ref, out_ref):
    pre = lax.dot_general(
        fc_ref[...], w_ref[...],
        (((0,), (0,)), ((), ())),
        preferred_element_type=jnp.float32,
    )
    out_ref[...] = jnp.tanh(pre)


@jax.jit
def kernel(feat, btype, Ws, bs, Wg, bg, Wl, bl):
    n, _ = feat.shape
    d = Ws.shape[1]
    wcat = jnp.zeros((16, d), jnp.float32)
    wcat = wcat.at[0:2].set(Ws).at[2:4].set(Wg).at[4:6].set(Wl)
    wcat = wcat.at[6].set(bs).at[7].set(bg).at[8].set(bl)

    feat_pad = jnp.pad(feat, ((0, _NPAD - n), (0, 0)))
    bt_pad = jnp.pad(btype, (0, _NPAD - n))

    fct = _route(feat_pad[:, 0], feat_pad[:, 1], bt_pad)

    return pl.pallas_call(
        _dense_body,
        grid=((n + _BLK - 1) // _BLK,),
        in_specs=[
            pl.BlockSpec((16, _BLK), lambda i: (0, i)),
            pl.BlockSpec((16, d), lambda i: (0, 0)),
        ],
        out_specs=pl.BlockSpec((_BLK, d), lambda i: (i, 0)),
        out_shape=jax.ShapeDtypeStruct((n, d), jnp.float32),
    )(fct, wcat)


# speedup vs baseline: 3.7128x; 1.1108x over previous
import functools

import jax
import jax.numpy as jnp
from jax import lax
from jax.experimental import pallas as pl
from jax.experimental.pallas import tpu as pltpu
from jax.experimental.pallas import tpu_sc as plsc

_NW = 32
_SLAB = 3200
_NPAD = _NW * _SLAB
_GROUPS = _SLAB // 16
_BLK = 4096


def _route_body(f0_hbm, f1_hbm, bt_hbm, out_hbm, f0_v, f1_v, bt_v, buf_v):
    wid = lax.axis_index("s") * 2 + lax.axis_index("c")
    base = wid * _SLAB
    pltpu.sync_copy(f0_hbm.at[pl.ds(base, _SLAB)], f0_v)
    pltpu.sync_copy(f1_hbm.at[pl.ds(base, _SLAB)], f1_v)
    pltpu.sync_copy(bt_hbm.at[pl.ds(base, _SLAB)], bt_v)

    zero_f = jnp.zeros((16,), jnp.float32)
    one_f = jnp.ones((16,), jnp.float32)

    def group(k, carry):
        r = k * 16
        tv = bt_v[pl.ds(r, 16)]
        f0 = f0_v[pl.ds(r, 16)]
        f1 = f1_v[pl.ds(r, 16)]
        m1 = tv == 1
        m2 = tv == 2
        m3 = tv == 3
        buf_v[0, pl.ds(r, 16)] = jnp.where(m1, f0, zero_f)
        buf_v[1, pl.ds(r, 16)] = jnp.where(m1, f1, zero_f)
        buf_v[2, pl.ds(r, 16)] = jnp.where(m2, f0, zero_f)
        buf_v[3, pl.ds(r, 16)] = jnp.where(m2, f1, zero_f)
        buf_v[4, pl.ds(r, 16)] = jnp.where(m3, f0, zero_f)
        buf_v[5, pl.ds(r, 16)] = jnp.where(m3, f1, zero_f)
        buf_v[6, pl.ds(r, 16)] = jnp.where(m1, one_f, zero_f)
        buf_v[7, pl.ds(r, 16)] = jnp.where(m2, one_f, zero_f)
        buf_v[8, pl.ds(r, 16)] = jnp.where(m3, one_f, zero_f)
        return carry

    lax.fori_loop(0, _GROUPS, group, 0)
    pltpu.sync_copy(buf_v, out_hbm.at[:, pl.ds(base, _SLAB)])


def _route(f0a, f1a, bt_pad):
    mesh = plsc.VectorSubcoreMesh(core_axis_name="c", subcore_axis_name="s")
    fn = functools.partial(
        pl.kernel,
        mesh=mesh,
        out_type=jax.ShapeDtypeStruct((9, _NPAD), jnp.float32),
        scratch_types=[
            pltpu.VMEM((_SLAB,), jnp.float32),
            pltpu.VMEM((_SLAB,), jnp.float32),
            pltpu.VMEM((_SLAB,), jnp.int32),
            pltpu.VMEM((9, _SLAB), jnp.float32),
        ],
    )(_route_body)
    return fn(f0a, f1a, bt_pad)


def _dense_body(fc_ref, w_ref, out_ref):
    pre = lax.dot_general(
        fc_ref[...], w_ref[...],
        (((0,), (0,)), ((), ())),
        preferred_element_type=jnp.float32,
    )
    out_ref[...] = jnp.tanh(pre)


@jax.jit
def kernel(feat, btype, Ws, bs, Wg, bg, Wl, bl):
    n, _ = feat.shape
    d = Ws.shape[1]
    wcat = jnp.zeros((9, d), jnp.float32)
    wcat = wcat.at[0:2].set(Ws).at[2:4].set(Wg).at[4:6].set(Wl)
    wcat = wcat.at[6].set(bs).at[7].set(bg).at[8].set(bl)

    feat_pad = jnp.pad(feat, ((0, _NPAD - n), (0, 0)))
    bt_pad = jnp.pad(btype, (0, _NPAD - n))

    fct = _route(feat_pad[:, 0], feat_pad[:, 1], bt_pad)

    return pl.pallas_call(
        _dense_body,
        grid=((n + _BLK - 1) // _BLK,),
        in_specs=[
            pl.BlockSpec((9, _BLK), lambda i: (0, i)),
            pl.BlockSpec((9, d), lambda i: (0, 0)),
        ],
        out_specs=pl.BlockSpec((_BLK, d), lambda i: (i, 0)),
        out_shape=jax.ShapeDtypeStruct((n, d), jnp.float32),
    )(fct, wcat)
